# baseline (speedup 1.0000x reference)
import functools

import jax
import jax.numpy as jnp
from jax.experimental import pallas as pl
from jax.experimental.pallas import tpu as pltpu

_N = 10000
_D = 128
_BLOCK = 5000


def _fused_kernel(x_ref, wi_ref, wc_ref, wo_ref, lw_ref,
                  bi_ref, bc_ref, bo_ref, wco_ref, lb_ref, out_ref):
    xb = x_ref[...]
    gi = jnp.dot(xb, wi_ref[...], preferred_element_type=jnp.float32)
    gc = jnp.dot(xb, wc_ref[...], preferred_element_type=jnp.float32)
    go = jnp.dot(xb, wo_ref[...], preferred_element_type=jnp.float32)
    I = jax.nn.sigmoid(gi + bi_ref[...])
    T = jnp.tanh(gc + bc_ref[...])
    C = I * T
    O = jax.nn.sigmoid(go + bo_ref[...] + wco_ref[...] * C)
    h = jnp.maximum(O * jnp.tanh(C), 0.0)
    out_ref[...] = (
        jnp.dot(h, lw_ref[...], preferred_element_type=jnp.float32)
        + lb_ref[...]
    )


@functools.partial(jax.jit, static_argnames=())
def _run(x, W_i, W_c, W_o, lin_W, bi, bc, bo, wc_o, lb):
    grid = (_N // _BLOCK,)
    row_spec = pl.BlockSpec((_BLOCK, _D), lambda i: (i, 0))
    w_spec = pl.BlockSpec((_D, _D), lambda i: (0, 0))
    b_spec = pl.BlockSpec((1, _D), lambda i: (0, 0))
    return pl.pallas_call(
        _fused_kernel,
        grid=grid,
        in_specs=[row_spec, w_spec, w_spec, w_spec, w_spec,
                  b_spec, b_spec, b_spec, b_spec, b_spec],
        out_specs=row_spec,
        out_shape=jax.ShapeDtypeStruct((_N, _D), jnp.float32),
        compiler_params=pltpu.CompilerParams(
            dimension_semantics=("parallel",),
        ),
    )(x, W_i, W_c, W_o, lin_W, bi, bc, bo, wc_o, lb)


def kernel(x, edge_index, edge_weight, W_i, b_i, convW_i, convb_i,
           W_f, b_f, convW_f, convb_f, W_c, b_c, convW_c, convb_c,
           W_o, b_o, convW_o, convb_o, wc_i, wc_f, wc_o, lin_W, lin_b):
    bi = b_i + convb_i[None, :]
    bc = b_c + convb_c[None, :]
    bo = b_o + convb_o[None, :]
    lb = lin_b[None, :]
    return _run(x, W_i, W_c, W_o, lin_W, bi, bc, bo, wc_o, lb)


# speedup vs baseline: 1.0237x; 1.0237x over previous
import functools

import jax
import jax.numpy as jnp
from jax.experimental import pallas as pl
from jax.experimental.pallas import tpu as pltpu

_N = 10000
_D = 128
_BLOCK = 2000


def _fused_kernel(x_ref, wi_ref, wc_ref, wo_ref, lw_ref,
                  bi_ref, bc_ref, bo_ref, wco_ref, lb_ref, out_ref):
    xb = x_ref[...]
    gi = jnp.dot(xb, wi_ref[...], preferred_element_type=jnp.float32)
    gc = jnp.dot(xb, wc_ref[...], preferred_element_type=jnp.float32)
    go = jnp.dot(xb, wo_ref[...], preferred_element_type=jnp.float32)
    I = jax.nn.sigmoid(gi + bi_ref[...])
    T = jnp.tanh(gc + bc_ref[...])
    C = I * T
    O = jax.nn.sigmoid(go + bo_ref[...] + wco_ref[...] * C)
    h = jnp.maximum(O * jnp.tanh(C), 0.0)
    out_ref[...] = (
        jnp.dot(h, lw_ref[...], preferred_element_type=jnp.float32)
        + lb_ref[...]
    )


@functools.partial(jax.jit, static_argnames=())
def _run(x, W_i, W_c, W_o, lin_W, bi, bc, bo, wc_o, lb):
    grid = (_N // _BLOCK,)
    row_spec = pl.BlockSpec((_BLOCK, _D), lambda i: (i, 0))
    w_spec = pl.BlockSpec((_D, _D), lambda i: (0, 0))
    b_spec = pl.BlockSpec((1, _D), lambda i: (0, 0))
    return pl.pallas_call(
        _fused_kernel,
        grid=grid,
        in_specs=[row_spec, w_spec, w_spec, w_spec, w_spec,
                  b_spec, b_spec, b_spec, b_spec, b_spec],
        out_specs=row_spec,
        out_shape=jax.ShapeDtypeStruct((_N, _D), jnp.float32),
        compiler_params=pltpu.CompilerParams(
            dimension_semantics=("parallel",),
        ),
    )(x, W_i, W_c, W_o, lin_W, bi, bc, bo, wc_o, lb)


def kernel(x, edge_index, edge_weight, W_i, b_i, convW_i, convb_i,
           W_f, b_f, convW_f, convb_f, W_c, b_c, convW_c, convb_c,
           W_o, b_o, convW_o, convb_o, wc_i, wc_f, wc_o, lin_W, lin_b):
    bi = b_i + convb_i[None, :]
    bc = b_c + convb_c[None, :]
    bo = b_o + convb_o[None, :]
    lb = lin_b[None, :]
    return _run(x, W_i, W_c, W_o, lin_W, bi, bc, bo, wc_o, lb)


# speedup vs baseline: 1.0257x; 1.0019x over previous
import functools

import jax
import jax.numpy as jnp
from jax.experimental import pallas as pl
from jax.experimental.pallas import tpu as pltpu

_N = 10000
_D = 128
_BLOCK = 2000


def _fused_kernel(x_ref, wi_ref, wc_ref, wo_ref, lw_ref,
                  bi_ref, bc_ref, bo_ref, wco_ref, lb_ref, out_ref):
    xb = x_ref[...]
    gi = jnp.dot(xb, wi_ref[...], preferred_element_type=jnp.float32)
    gc = jnp.dot(xb, wc_ref[...], preferred_element_type=jnp.float32)
    go = jnp.dot(xb, wo_ref[...], preferred_element_type=jnp.float32)
    I = jax.nn.sigmoid(gi + bi_ref[...])
    T = jnp.tanh(gc + bc_ref[...])
    C = I * T
    O = jax.nn.sigmoid(go + bo_ref[...] + wco_ref[...] * C)
    h = jnp.maximum(O * jnp.tanh(C), 0.0)
    out_ref[...] = (
        jnp.dot(h, lw_ref[...], preferred_element_type=jnp.float32)
        + lb_ref[...]
    )


@functools.partial(jax.jit, static_argnames=())
def _run(x, W_i, W_c, W_o, lin_W, bi, bc, bo, wc_o, lb):
    grid = (_N // _BLOCK,)
    row_spec = pl.BlockSpec((_BLOCK, _D), lambda i: (i, 0))
    w_spec = pl.BlockSpec((_D, _D), lambda i: (0, 0))
    b_spec = pl.BlockSpec((1, _D), lambda i: (0, 0))
    return pl.pallas_call(
        _fused_kernel,
        grid=grid,
        in_specs=[row_spec, w_spec, w_spec, w_spec, w_spec,
                  b_spec, b_spec, b_spec, b_spec, b_spec],
        out_specs=row_spec,
        out_shape=jax.ShapeDtypeStruct((_N, _D), jnp.float32),
        compiler_params=pltpu.CompilerParams(
            dimension_semantics=("arbitrary",),
        ),
    )(x, W_i, W_c, W_o, lin_W, bi, bc, bo, wc_o, lb)


def kernel(x, edge_index, edge_weight, W_i, b_i, convW_i, convb_i,
           W_f, b_f, convW_f, convb_f, W_c, b_c, convW_c, convb_c,
           W_o, b_o, convW_o, convb_o, wc_i, wc_f, wc_o, lin_W, lin_b):
    bi = b_i + convb_i[None, :]
    bc = b_c + convb_c[None, :]
    bo = b_o + convb_o[None, :]
    lb = lin_b[None, :]
    return _run(x, W_i, W_c, W_o, lin_W, bi, bc, bo, wc_o, lb)


# speedup vs baseline: 1.2834x; 1.2513x over previous
import functools

import jax
import jax.numpy as jnp
from jax.experimental import pallas as pl
from jax.experimental.pallas import tpu as pltpu

_N = 10000
_D = 128
_BLOCK = 2000


def _fused_kernel(x_ref, wi_ref, wc_ref, wo_ref, lw_ref,
                  bi_ref, cbi_ref, bc_ref, cbc_ref, bo_ref, cbo_ref,
                  wco_ref, lb_ref, out_ref):
    xb = x_ref[...]
    gi = jnp.dot(xb, wi_ref[...], preferred_element_type=jnp.float32)
    gc = jnp.dot(xb, wc_ref[...], preferred_element_type=jnp.float32)
    go = jnp.dot(xb, wo_ref[...], preferred_element_type=jnp.float32)
    I = jax.nn.sigmoid(gi + (bi_ref[...] + cbi_ref[...]))
    T = jnp.tanh(gc + (bc_ref[...] + cbc_ref[...]))
    C = I * T
    O = jax.nn.sigmoid(go + (bo_ref[...] + cbo_ref[...]) + wco_ref[...] * C)
    h = jnp.maximum(O * jnp.tanh(C), 0.0)
    out_ref[...] = (
        jnp.dot(h, lw_ref[...], preferred_element_type=jnp.float32)
        + lb_ref[...]
    )


@functools.partial(jax.jit, static_argnames=())
def _run(x, W_i, W_c, W_o, lin_W, b_i, convb_i, b_c, convb_c, b_o, convb_o,
         wc_o, lin_b):
    grid = (_N // _BLOCK,)
    row_spec = pl.BlockSpec((_BLOCK, _D), lambda i: (i, 0))
    w_spec = pl.BlockSpec((_D, _D), lambda i: (0, 0))
    b_spec = pl.BlockSpec((1, _D), lambda i: (0, 0))
    return pl.pallas_call(
        _fused_kernel,
        grid=grid,
        in_specs=[row_spec, w_spec, w_spec, w_spec, w_spec,
                  b_spec, b_spec, b_spec, b_spec, b_spec, b_spec,
                  b_spec, b_spec],
        out_specs=row_spec,
        out_shape=jax.ShapeDtypeStruct((_N, _D), jnp.float32),
        compiler_params=pltpu.CompilerParams(
            dimension_semantics=("arbitrary",),
        ),
    )(x, W_i, W_c, W_o, lin_W, b_i, convb_i.reshape(1, _D),
      b_c, convb_c.reshape(1, _D), b_o, convb_o.reshape(1, _D),
      wc_o, lin_b.reshape(1, _D))


def kernel(x, edge_index, edge_weight, W_i, b_i, convW_i, convb_i,
           W_f, b_f, convW_f, convb_f, W_c, b_c, convW_c, convb_c,
           W_o, b_o, convW_o, convb_o, wc_i, wc_f, wc_o, lin_W, lin_b):
    return _run(x, W_i, W_c, W_o, lin_W, b_i, convb_i, b_c, convb_c,
                b_o, convb_o, wc_o, lin_b)
